# baseline (speedup 1.0000x reference)
import functools

import jax
import jax.numpy as jnp
from jax import lax
from jax.experimental import pallas as pl
from jax.experimental.pallas import tpu as pltpu
from jax.experimental.pallas import tpu_sc as plsc

NW = 32
NS = 16
CK = 128
NBUF = 3
DEPTH = 2
BM = 1000


@functools.lru_cache(maxsize=None)
def _sc_agg_fn(n, dh, ch, n_pad):
    mesh = plsc.VectorSubcoreMesh(core_axis_name="c", subcore_axis_name="s")
    nz = n_pad // NS
    ns = n // NS

    def body(y_hbm, src_hbm, dst_hbm, z_hbm, out_hbm, src_v, dst_v, buf,
             ystage, shared, gsem, ssem):
        cid = lax.axis_index("c")
        sid = lax.axis_index("s")
        wid = cid * NS + sid
        pltpu.sync_copy(src_hbm.at[wid], src_v)
        pltpu.sync_copy(dst_hbm.at[wid], dst_v)
        pltpu.sync_copy(y_hbm.at[cid].at[pl.ds(sid * ns, ns)],
                        ystage.at[pl.ds(sid * ns, ns)])
        pltpu.sync_copy(z_hbm.at[pl.ds(sid * nz, nz)],
                        shared.at[pl.ds(sid * nz, nz)])
        plsc.subcore_barrier()

        for r in range(DEPTH):
            pltpu.async_copy(ystage.at[src_v.at[r]], buf.at[r], gsem.at[r])

        def step(j, carry):
            s = lax.rem(j, NBUF)
            pltpu.make_async_copy(ystage.at[src_v.at[j]], buf.at[s],
                                  gsem.at[s]).wait()
            pltpu.async_copy(buf.at[s], shared.at[dst_v.at[j]], ssem.at[s],
                             add=True)
            jj = j + DEPTH

            @pl.when(jj < ch)
            def _():
                sj = lax.rem(jj, NBUF)

                @pl.when(jj >= NBUF)
                def _():
                    pltpu.make_async_copy(buf.at[sj],
                                          shared.at[dst_v.at[jj - NBUF]],
                                          ssem.at[sj]).wait()

                pltpu.async_copy(ystage.at[src_v.at[jj]], buf.at[sj],
                                 gsem.at[sj])

            return carry

        lax.fori_loop(0, ch, step, 0)
        for r in range(NBUF):
            pltpu.make_async_copy(buf.at[r], shared.at[dst_v.at[0]],
                                  ssem.at[r]).wait()
        plsc.subcore_barrier()
        pltpu.sync_copy(shared.at[pl.ds(sid * nz, nz)],
                        out_hbm.at[cid].at[pl.ds(sid * nz, nz)])

    return pl.kernel(
        body,
        out_type=jax.ShapeDtypeStruct((2, n_pad, dh), jnp.float32),
        mesh=mesh,
        scratch_types=[
            pltpu.VMEM((ch, CK), jnp.int32),
            pltpu.VMEM((ch, CK), jnp.int32),
            pltpu.VMEM((NBUF, CK, dh), jnp.float32),
            pltpu.VMEM_SHARED((n, dh), jnp.float32),
            pltpu.VMEM_SHARED((n_pad, dh), jnp.float32),
            pltpu.SemaphoreType.DMA((NBUF,)),
            pltpu.SemaphoreType.DMA((NBUF,)),
        ],
        compiler_params=pltpu.CompilerParams(use_tc_tiling_on_sc=False),
    )


def _sc_agg(y2, src_r, dst_r, zeros):
    _, n, dh = y2.shape
    ch = src_r.shape[1]
    return _sc_agg_fn(n, dh, ch, zeros.shape[0])(y2, src_r, dst_r, zeros)


def _mm_first(x, w_self, w_neigh, b):
    m, d = x.shape
    dh = d // 2

    def body(x_ref, ws_ref, wn_ref, b_ref, xs_ref, y_ref):
        xb = x_ref[...]
        xs_ref[...] = (jnp.dot(xb, ws_ref[...], preferred_element_type=jnp.float32)
                       + b_ref[...])
        y = jnp.dot(xb, wn_ref[...], preferred_element_type=jnp.float32)
        y_ref[0] = y[:, :dh]
        y_ref[1] = y[:, dh:]

    return pl.pallas_call(
        body,
        grid=(m // BM,),
        in_specs=[
            pl.BlockSpec((BM, d), lambda i: (i, 0)),
            pl.BlockSpec((d, d), lambda i: (0, 0)),
            pl.BlockSpec((d, d), lambda i: (0, 0)),
            pl.BlockSpec((1, d), lambda i: (0, 0)),
        ],
        out_specs=[pl.BlockSpec((BM, d), lambda i: (i, 0)),
                   pl.BlockSpec((2, BM, dh), lambda i: (0, i, 0))],
        out_shape=[jax.ShapeDtypeStruct((m, d), jnp.float32),
                   jax.ShapeDtypeStruct((2, m, dh), jnp.float32)],
    )(x, w_self, w_neigh, b.reshape(1, d))


def _mm_mid(xs_prev, agg, w_self, w_neigh, b, res=None):
    m, d = xs_prev.shape
    dh = d // 2
    with_res = res is not None

    def body(*refs):
        if with_res:
            xsp_ref, agg_ref, ws_ref, wn_ref, b_ref, r_ref, xs_ref, y_ref, x_ref = refs
        else:
            xsp_ref, agg_ref, ws_ref, wn_ref, b_ref, xs_ref, y_ref, x_ref = refs
        a = jnp.concatenate([agg_ref[0], agg_ref[1]], axis=1)
        x = jnp.maximum(xsp_ref[...] + a, 0.0)
        if with_res:
            x = x + r_ref[...]
        x_ref[...] = x
        xs_ref[...] = (jnp.dot(x, ws_ref[...], preferred_element_type=jnp.float32)
                       + b_ref[...])
        y = jnp.dot(x, wn_ref[...], preferred_element_type=jnp.float32)
        y_ref[0] = y[:, :dh]
        y_ref[1] = y[:, dh:]

    in_specs = [
        pl.BlockSpec((BM, d), lambda i: (i, 0)),
        pl.BlockSpec((2, BM, dh), lambda i: (0, i, 0)),
        pl.BlockSpec((d, d), lambda i: (0, 0)),
        pl.BlockSpec((d, d), lambda i: (0, 0)),
        pl.BlockSpec((1, d), lambda i: (0, 0)),
    ]
    args = [xs_prev, agg, w_self, w_neigh, b.reshape(1, d)]
    if with_res:
        in_specs.append(pl.BlockSpec((BM, d), lambda i: (i, 0)))
        args.append(res)
    return pl.pallas_call(
        body,
        grid=(m // BM,),
        in_specs=in_specs,
        out_specs=[pl.BlockSpec((BM, d), lambda i: (i, 0)),
                   pl.BlockSpec((2, BM, dh), lambda i: (0, i, 0)),
                   pl.BlockSpec((BM, d), lambda i: (i, 0))],
        out_shape=[jax.ShapeDtypeStruct((m, d), jnp.float32),
                   jax.ShapeDtypeStruct((2, m, dh), jnp.float32),
                   jax.ShapeDtypeStruct((m, d), jnp.float32)],
    )(*args)


def _combine_final(xs, agg):
    m, d = xs.shape
    dh = d // 2

    def body(xs_ref, agg_ref, o_ref):
        a = jnp.concatenate([agg_ref[0], agg_ref[1]], axis=1)
        o_ref[...] = jnp.maximum(xs_ref[...] + a, 0.0)

    return pl.pallas_call(
        body,
        grid=(m // BM,),
        in_specs=[pl.BlockSpec((BM, d), lambda i: (i, 0)),
                  pl.BlockSpec((2, BM, dh), lambda i: (0, i, 0))],
        out_specs=pl.BlockSpec((BM, d), lambda i: (i, 0)),
        out_shape=jax.ShapeDtypeStruct((m, d), jnp.float32),
    )(xs, agg)


def kernel(features, edges, Ws, Wn, bs):
    n, d = features.shape
    e = edges.shape[1]
    ch = -(-e // (NW * CK))
    ch = -(-ch // NBUF) * NBUF
    e_pad = NW * ch * CK
    n_pad = ((n + 1 + 127) // 128) * 128

    src = edges[0]
    dst = edges[1]
    pad = e_pad - e
    src_r = jnp.concatenate([src, jnp.zeros((pad,), jnp.int32)]).reshape(NW, ch, CK)
    dst_r = jnp.concatenate([dst, jnp.full((pad,), n, jnp.int32)]).reshape(NW, ch, CK)
    zeros = jnp.zeros((n_pad, d // 2), jnp.float32)

    xs, y2 = _mm_first(features, Ws[0], Wn[0], bs[0])
    agg = _sc_agg(y2, src_r, dst_r, zeros)
    h0 = None
    for l in range(1, 13):
        xs, y2, x = _mm_mid(xs, agg, Ws[l], Wn[l], bs[l])
        if l == 1:
            h0 = x
        agg = _sc_agg(y2, src_r, dst_r, zeros)
    xs, y2, _ = _mm_mid(xs, agg, Ws[13], Wn[13], bs[13], res=h0)
    agg = _sc_agg(y2, src_r, dst_r, zeros)
    return _combine_final(xs, agg)


# speedup vs baseline: 1.0226x; 1.0226x over previous
import functools

import jax
import jax.numpy as jnp
from jax import lax
from jax.experimental import pallas as pl
from jax.experimental.pallas import tpu as pltpu
from jax.experimental.pallas import tpu_sc as plsc

NW = 32
NS = 16
CK = 64
NBUF = 6
DEPTH = 3
BM = 1000


@functools.lru_cache(maxsize=None)
def _sc_agg_fn(n, dh, ch, n_pad):
    mesh = plsc.VectorSubcoreMesh(core_axis_name="c", subcore_axis_name="s")
    nz = n_pad // NS
    ns = n // NS

    def body(y_hbm, src_hbm, dst_hbm, z_hbm, out_hbm, src_v, dst_v, buf,
             ystage, shared, gsem, ssem):
        cid = lax.axis_index("c")
        sid = lax.axis_index("s")
        wid = cid * NS + sid
        pltpu.sync_copy(src_hbm.at[wid], src_v)
        pltpu.sync_copy(dst_hbm.at[wid], dst_v)
        pltpu.sync_copy(y_hbm.at[cid].at[pl.ds(sid * ns, ns)],
                        ystage.at[pl.ds(sid * ns, ns)])
        pltpu.sync_copy(z_hbm.at[pl.ds(sid * nz, nz)],
                        shared.at[pl.ds(sid * nz, nz)])
        plsc.subcore_barrier()

        for r in range(DEPTH):
            pltpu.async_copy(ystage.at[src_v.at[r]], buf.at[r], gsem.at[r])

        def step(j, carry):
            s = lax.rem(j, NBUF)
            pltpu.make_async_copy(ystage.at[src_v.at[j]], buf.at[s],
                                  gsem.at[s]).wait()
            pltpu.async_copy(buf.at[s], shared.at[dst_v.at[j]], ssem.at[s],
                             add=True)
            jj = j + DEPTH

            @pl.when(jj < ch)
            def _():
                sj = lax.rem(jj, NBUF)

                @pl.when(jj >= NBUF)
                def _():
                    pltpu.make_async_copy(buf.at[sj],
                                          shared.at[dst_v.at[jj - NBUF]],
                                          ssem.at[sj]).wait()

                pltpu.async_copy(ystage.at[src_v.at[jj]], buf.at[sj],
                                 gsem.at[sj])

            return carry

        lax.fori_loop(0, ch, step, 0)
        for r in range(NBUF):
            pltpu.make_async_copy(buf.at[r], shared.at[dst_v.at[0]],
                                  ssem.at[r]).wait()
        plsc.subcore_barrier()
        pltpu.sync_copy(shared.at[pl.ds(sid * nz, nz)],
                        out_hbm.at[cid].at[pl.ds(sid * nz, nz)])

    return pl.kernel(
        body,
        out_type=jax.ShapeDtypeStruct((2, n_pad, dh), jnp.float32),
        mesh=mesh,
        scratch_types=[
            pltpu.VMEM((ch, CK), jnp.int32),
            pltpu.VMEM((ch, CK), jnp.int32),
            pltpu.VMEM((NBUF, CK, dh), jnp.float32),
            pltpu.VMEM_SHARED((n, dh), jnp.float32),
            pltpu.VMEM_SHARED((n_pad, dh), jnp.float32),
            pltpu.SemaphoreType.DMA((NBUF,)),
            pltpu.SemaphoreType.DMA((NBUF,)),
        ],
        compiler_params=pltpu.CompilerParams(use_tc_tiling_on_sc=False),
    )


def _sc_agg(y2, src_r, dst_r, zeros):
    _, n, dh = y2.shape
    ch = src_r.shape[1]
    return _sc_agg_fn(n, dh, ch, zeros.shape[0])(y2, src_r, dst_r, zeros)


def _mm_self(xsp, w_self, b):
    _, m, dh = xsp.shape
    d = 2 * dh

    def body(x_ref, ws_ref, b_ref, xs_ref):
        x = jnp.concatenate([x_ref[0], x_ref[1]], axis=1)
        xs_ref[...] = (jnp.dot(x, ws_ref[...], preferred_element_type=jnp.float32)
                       + b_ref[...])

    return pl.pallas_call(
        body,
        grid=(m // BM,),
        in_specs=[pl.BlockSpec((2, BM, dh), lambda i: (0, i, 0)),
                  pl.BlockSpec((d, d), lambda i: (0, 0)),
                  pl.BlockSpec((1, d), lambda i: (0, 0))],
        out_specs=pl.BlockSpec((BM, d), lambda i: (i, 0)),
        out_shape=jax.ShapeDtypeStruct((m, d), jnp.float32),
    )(xsp, w_self, b.reshape(1, d))


def _combine(xs, agg, w_neigh, res=None, split=True):
    m, d = xs.shape
    dh = d // 2
    with_res = res is not None

    def body(*refs):
        if with_res:
            xs_ref, agg_ref, wn_ref, r_ref, o_ref = refs
        else:
            xs_ref, agg_ref, wn_ref, o_ref = refs
        a = jnp.concatenate([agg_ref[0], agg_ref[1]], axis=1)
        x = jnp.maximum(
            xs_ref[...]
            + jnp.dot(a, wn_ref[...], preferred_element_type=jnp.float32), 0.0)
        if with_res:
            x = jnp.concatenate([r_ref[0], r_ref[1]], axis=1) + x
        if split:
            o_ref[0] = x[:, :dh]
            o_ref[1] = x[:, dh:]
        else:
            o_ref[...] = x

    in_specs = [
        pl.BlockSpec((BM, d), lambda i: (i, 0)),
        pl.BlockSpec((2, BM, dh), lambda i: (0, i, 0)),
        pl.BlockSpec((d, d), lambda i: (0, 0)),
    ]
    args = [xs, agg, w_neigh]
    if with_res:
        in_specs.append(pl.BlockSpec((2, BM, dh), lambda i: (0, i, 0)))
        args.append(res)
    if split:
        out_spec = pl.BlockSpec((2, BM, dh), lambda i: (0, i, 0))
        out_shape = jax.ShapeDtypeStruct((2, m, dh), jnp.float32)
    else:
        out_spec = pl.BlockSpec((BM, d), lambda i: (i, 0))
        out_shape = jax.ShapeDtypeStruct((m, d), jnp.float32)
    return pl.pallas_call(
        body,
        grid=(m // BM,),
        in_specs=in_specs,
        out_specs=out_spec,
        out_shape=out_shape,
    )(*args)


def _split_rows(x):
    m, d = x.shape
    dh = d // 2

    def body(x_ref, o_ref):
        o_ref[0] = x_ref[..., :dh]
        o_ref[1] = x_ref[..., dh:]

    return pl.pallas_call(
        body,
        grid=(m // BM,),
        in_specs=[pl.BlockSpec((BM, d), lambda i: (i, 0))],
        out_specs=pl.BlockSpec((2, BM, dh), lambda i: (0, i, 0)),
        out_shape=jax.ShapeDtypeStruct((2, m, dh), jnp.float32),
    )(x)


def kernel(features, edges, Ws, Wn, bs):
    n, d = features.shape
    e = edges.shape[1]
    ch = -(-e // (NW * CK))
    ch = -(-ch // NBUF) * NBUF
    e_pad = NW * ch * CK
    n_pad = ((n + 1 + 127) // 128) * 128

    src = edges[0]
    dst = edges[1]
    pad = e_pad - e
    src_r = jnp.concatenate([src, jnp.zeros((pad,), jnp.int32)]).reshape(NW, ch, CK)
    dst_r = jnp.concatenate([dst, jnp.full((pad,), n, jnp.int32)]).reshape(NW, ch, CK)
    zeros = jnp.zeros((n_pad, d // 2), jnp.float32)

    xsp = _split_rows(features)
    h0 = None
    for l in range(14):
        agg = _sc_agg(xsp, src_r, dst_r, zeros)
        xs = _mm_self(xsp, Ws[l], bs[l])
        last = l == 13
        xsp = _combine(xs, agg, Wn[l],
                       res=h0 if l == 12 else None, split=not last)
        if l == 0:
            h0 = xsp
    return xsp


# speedup vs baseline: 1.0424x; 1.0194x over previous
import functools

import jax
import jax.numpy as jnp
from jax import lax
from jax.experimental import pallas as pl
from jax.experimental.pallas import tpu as pltpu
from jax.experimental.pallas import tpu_sc as plsc

NW = 32
NS = 16
CK = 64
NBUF = 6
DEPTH = 3
BM = 1000


@functools.lru_cache(maxsize=None)
def _sc_agg_fn(n, dh, ch, n_pad):
    mesh = plsc.VectorSubcoreMesh(core_axis_name="c", subcore_axis_name="s")
    nz = n_pad // NS
    ns = n // NS

    def body(y_hbm, src_hbm, dst_hbm, z_hbm, out_hbm, src_v, dst_v, buf,
             ystage, shared, gsem, ssem):
        cid = lax.axis_index("c")
        sid = lax.axis_index("s")
        wid = cid * NS + sid
        d0 = pltpu.async_copy(src_hbm.at[wid], src_v, gsem.at[0])
        d1 = pltpu.async_copy(dst_hbm.at[wid], dst_v, gsem.at[1])
        d2 = pltpu.async_copy(y_hbm.at[cid].at[pl.ds(sid * ns, ns)],
                              ystage.at[pl.ds(sid * ns, ns)], gsem.at[2])
        d3 = pltpu.async_copy(z_hbm.at[pl.ds(sid * nz, nz)],
                              shared.at[pl.ds(sid * nz, nz)], gsem.at[3])
        d0.wait()
        d1.wait()
        d2.wait()
        d3.wait()
        plsc.subcore_barrier()

        for r in range(DEPTH):
            pltpu.async_copy(ystage.at[src_v.at[r]], buf.at[r], gsem.at[r])

        def step(j, carry):
            s = lax.rem(j, NBUF)
            pltpu.make_async_copy(ystage.at[src_v.at[j]], buf.at[s],
                                  gsem.at[s]).wait()
            pltpu.async_copy(buf.at[s], shared.at[dst_v.at[j]], ssem.at[s],
                             add=True)
            jj = j + DEPTH

            @pl.when(jj < ch)
            def _():
                sj = lax.rem(jj, NBUF)

                @pl.when(jj >= NBUF)
                def _():
                    pltpu.make_async_copy(buf.at[sj],
                                          shared.at[dst_v.at[jj - NBUF]],
                                          ssem.at[sj]).wait()

                pltpu.async_copy(ystage.at[src_v.at[jj]], buf.at[sj],
                                 gsem.at[sj])

            return carry

        lax.fori_loop(0, ch, step, 0)
        for r in range(NBUF):
            pltpu.make_async_copy(buf.at[r], shared.at[dst_v.at[0]],
                                  ssem.at[r]).wait()
        plsc.subcore_barrier()
        pltpu.sync_copy(shared.at[pl.ds(sid * nz, nz)],
                        out_hbm.at[cid].at[pl.ds(sid * nz, nz)])

    return pl.kernel(
        body,
        out_type=jax.ShapeDtypeStruct((2, n_pad, dh), jnp.float32),
        mesh=mesh,
        scratch_types=[
            pltpu.VMEM((ch, CK), jnp.int32),
            pltpu.VMEM((ch, CK), jnp.int32),
            pltpu.VMEM((NBUF, CK, dh), jnp.float32),
            pltpu.VMEM_SHARED((n, dh), jnp.float32),
            pltpu.VMEM_SHARED((n_pad, dh), jnp.float32),
            pltpu.SemaphoreType.DMA((NBUF,)),
            pltpu.SemaphoreType.DMA((NBUF,)),
        ],
        compiler_params=pltpu.CompilerParams(use_tc_tiling_on_sc=False),
    )


def _sc_agg(y2, src_r, dst_r, zeros):
    _, n, dh = y2.shape
    ch = src_r.shape[1]
    return _sc_agg_fn(n, dh, ch, zeros.shape[0])(y2, src_r, dst_r, zeros)


def _mm_self(xsp, w_self, b):
    _, m, dh = xsp.shape
    d = 2 * dh

    def body(x_ref, ws_ref, b_ref, xs_ref):
        x = jnp.concatenate([x_ref[0], x_ref[1]], axis=1)
        xs_ref[...] = (jnp.dot(x, ws_ref[...], preferred_element_type=jnp.float32)
                       + b_ref[...])

    return pl.pallas_call(
        body,
        grid=(m // BM,),
        in_specs=[pl.BlockSpec((2, BM, dh), lambda i: (0, i, 0)),
                  pl.BlockSpec((d, d), lambda i: (0, 0)),
                  pl.BlockSpec((1, d), lambda i: (0, 0))],
        out_specs=pl.BlockSpec((BM, d), lambda i: (i, 0)),
        out_shape=jax.ShapeDtypeStruct((m, d), jnp.float32),
    )(xsp, w_self, b.reshape(1, d))


def _combine(xs, agg, w_neigh, res=None, split=True):
    m, d = xs.shape
    dh = d // 2
    with_res = res is not None

    def body(*refs):
        if with_res:
            xs_ref, agg_ref, wn_ref, r_ref, o_ref = refs
        else:
            xs_ref, agg_ref, wn_ref, o_ref = refs
        a = jnp.concatenate([agg_ref[0], agg_ref[1]], axis=1)
        x = jnp.maximum(
            xs_ref[...]
            + jnp.dot(a, wn_ref[...], preferred_element_type=jnp.float32), 0.0)
        if with_res:
            x = jnp.concatenate([r_ref[0], r_ref[1]], axis=1) + x
        if split:
            o_ref[0] = x[:, :dh]
            o_ref[1] = x[:, dh:]
        else:
            o_ref[...] = x

    in_specs = [
        pl.BlockSpec((BM, d), lambda i: (i, 0)),
        pl.BlockSpec((2, BM, dh), lambda i: (0, i, 0)),
        pl.BlockSpec((d, d), lambda i: (0, 0)),
    ]
    args = [xs, agg, w_neigh]
    if with_res:
        in_specs.append(pl.BlockSpec((2, BM, dh), lambda i: (0, i, 0)))
        args.append(res)
    if split:
        out_spec = pl.BlockSpec((2, BM, dh), lambda i: (0, i, 0))
        out_shape = jax.ShapeDtypeStruct((2, m, dh), jnp.float32)
    else:
        out_spec = pl.BlockSpec((BM, d), lambda i: (i, 0))
        out_shape = jax.ShapeDtypeStruct((m, d), jnp.float32)
    return pl.pallas_call(
        body,
        grid=(m // BM,),
        in_specs=in_specs,
        out_specs=out_spec,
        out_shape=out_shape,
    )(*args)


def _split_rows(x):
    m, d = x.shape
    dh = d // 2

    def body(x_ref, o_ref):
        o_ref[0] = x_ref[..., :dh]
        o_ref[1] = x_ref[..., dh:]

    return pl.pallas_call(
        body,
        grid=(m // BM,),
        in_specs=[pl.BlockSpec((BM, d), lambda i: (i, 0))],
        out_specs=pl.BlockSpec((2, BM, dh), lambda i: (0, i, 0)),
        out_shape=jax.ShapeDtypeStruct((2, m, dh), jnp.float32),
    )(x)


def kernel(features, edges, Ws, Wn, bs):
    n, d = features.shape
    e = edges.shape[1]
    ch = -(-e // (NW * CK))
    ch = -(-ch // NBUF) * NBUF
    e_pad = NW * ch * CK
    n_pad = ((n + 1 + 127) // 128) * 128

    src = edges[0]
    dst = edges[1]
    pad = e_pad - e
    src_r = jnp.concatenate([src, jnp.zeros((pad,), jnp.int32)]).reshape(NW, ch, CK)
    dst_r = jnp.concatenate([dst, jnp.full((pad,), n, jnp.int32)]).reshape(NW, ch, CK)
    zeros = jnp.zeros((n_pad, d // 2), jnp.float32)

    xsp = _split_rows(features)
    h0 = None
    for l in range(14):
        agg = _sc_agg(xsp, src_r, dst_r, zeros)
        xs = _mm_self(xsp, Ws[l], bs[l])
        last = l == 13
        xsp = _combine(xs, agg, Wn[l],
                       res=h0 if l == 12 else None, split=not last)
        if l == 0:
            h0 = xsp
    return xsp
